# baseline (speedup 1.0000x reference)
import jax
import jax.numpy as jnp
from jax.experimental import pallas as pl
from jax.experimental.pallas import tpu as pltpu

_CH = 8


def _body(mask_ref, out_ref, fng_ref, fnm_ref, oresi_ref, fpg_ref, fpm_ref,
          res_ref, acc_ref):
    step = pl.program_id(0)

    @pl.when(step == 0)
    def _init():
        for i in range(4):
            acc_ref[i] = 0.0

    def branch(o, g, m):
        om = 1.0 - o
        g2 = (1.0 - g) * (1.0 - g)
        g4 = g2 * g2
        is_neg = m == 0
        x = jnp.where(is_neg, om, o)
        w = jnp.where(is_neg, o * o * g4, om * om)
        t = jnp.log(x) * w
        t_neg = jnp.where(is_neg, t, 0.0)
        return jnp.sum(t_neg), jnp.sum(t)

    n_fn, t_fn = branch(out_ref[...], fng_ref[...], fnm_ref[...])
    n_fp, t_fp = branch(oresi_ref[...], fpg_ref[...], fpm_ref[...])
    acc_ref[0] = acc_ref[0] + n_fn
    acc_ref[1] = acc_ref[1] + t_fn
    acc_ref[2] = acc_ref[2] + n_fp
    acc_ref[3] = acc_ref[3] + t_fp

    @pl.when(step == pl.num_programs(0) - 1)
    def _fini():
        num_pos = jnp.sum(mask_ref[...])
        res_ref[0] = jnp.where(num_pos == 0.0, -acc_ref[0], -acc_ref[1])
        res_ref[1] = jnp.where(num_pos == 0.0, -acc_ref[2], -acc_ref[3])


def kernel(out, out_resi, target_resi, mask, negloss_fn_gt, fn_mask,
           negloss_fp_gt, fp_mask, wh_):
    B, C, H, W = out.shape
    BC = B * C
    view = lambda a: a.reshape(BC, H, W)
    grid = (BC // _CH,)
    big = pl.BlockSpec((_CH, H, W), lambda i: (i, 0, 0))
    res = pl.pallas_call(
        _body,
        grid=grid,
        in_specs=[
            pl.BlockSpec(mask.shape, lambda i: (0, 0)),
            big, big, big, big, big, big,
        ],
        out_specs=pl.BlockSpec(memory_space=pltpu.SMEM),
        out_shape=jax.ShapeDtypeStruct((2,), jnp.float32),
        scratch_shapes=[pltpu.SMEM((4,), jnp.float32)],
    )(mask, view(out), view(negloss_fn_gt), view(fn_mask),
      view(out_resi), view(negloss_fp_gt), view(fp_mask))
    return res[0], res[1]


# speedup vs baseline: 1.3500x; 1.3500x over previous
import jax
import jax.numpy as jnp
from jax.experimental import pallas as pl
from jax.experimental.pallas import tpu as pltpu

_ROWS = 2048
_CHUNK = 8


def _body(mask_ref, o_fn_ref, g_fn_ref, m_fn_ref, o_fp_ref, g_fp_ref,
          m_fp_ref, res_ref, acc_ref):
    step = pl.program_id(0)

    @pl.when(step == 0)
    def _init():
        for i in range(4):
            acc_ref[i] = 0.0

    def chunk(j, carry):
        tot_fn, neg_fn, tot_fp, neg_fp = carry
        r = j * _CHUNK

        def branch(o_ref, g_ref, m_ref):
            o = o_ref[pl.ds(r, _CHUNK), :]
            g = g_ref[pl.ds(r, _CHUNK), :]
            m = m_ref[pl.ds(r, _CHUNK), :]
            om = 1.0 - o
            g1 = 1.0 - g
            g2 = g1 * g1
            g4 = g2 * g2
            isn = m == 0
            x = jnp.where(isn, om, o)
            xm = jnp.where(isn, o, om)
            w = xm * xm * jnp.where(isn, g4, 1.0)
            t = jnp.log(x) * w
            tn = jnp.where(isn, t, 0.0)
            return t, tn

        t1, tn1 = branch(o_fn_ref, g_fn_ref, m_fn_ref)
        t2, tn2 = branch(o_fp_ref, g_fp_ref, m_fp_ref)
        return (tot_fn + t1, neg_fn + tn1, tot_fp + t2, neg_fp + tn2)

    z = jnp.zeros((_CHUNK, 128), jnp.float32)
    tot_fn, neg_fn, tot_fp, neg_fp = jax.lax.fori_loop(
        0, _ROWS // _CHUNK, chunk, (z, z, z, z), unroll=8)

    acc_ref[0] = acc_ref[0] + jnp.sum(neg_fn)
    acc_ref[1] = acc_ref[1] + jnp.sum(tot_fn)
    acc_ref[2] = acc_ref[2] + jnp.sum(neg_fp)
    acc_ref[3] = acc_ref[3] + jnp.sum(tot_fp)

    @pl.when(step == pl.num_programs(0) - 1)
    def _fini():
        num_pos = jnp.sum(mask_ref[...])
        res_ref[0] = jnp.where(num_pos == 0.0, -acc_ref[0], -acc_ref[1])
        res_ref[1] = jnp.where(num_pos == 0.0, -acc_ref[2], -acc_ref[3])


def kernel(out, out_resi, target_resi, mask, negloss_fn_gt, fn_mask,
           negloss_fp_gt, fp_mask, wh_):
    B, C, H, W = out.shape
    R = B * C * H
    view = lambda a: a.reshape(R, W)
    grid = (R // _ROWS,)
    big = pl.BlockSpec((_ROWS, W), lambda i: (i, 0))
    res = pl.pallas_call(
        _body,
        grid=grid,
        in_specs=[
            pl.BlockSpec(mask.shape, lambda i: (0, 0)),
            big, big, big, big, big, big,
        ],
        out_specs=pl.BlockSpec(memory_space=pltpu.SMEM),
        out_shape=jax.ShapeDtypeStruct((2,), jnp.float32),
        scratch_shapes=[pltpu.SMEM((4,), jnp.float32)],
    )(mask, view(out), view(negloss_fn_gt), view(fn_mask),
      view(out_resi), view(negloss_fp_gt), view(fp_mask))
    return res[0], res[1]


# speedup vs baseline: 1.4904x; 1.1040x over previous
import jax
import jax.numpy as jnp
from jax.experimental import pallas as pl
from jax.experimental.pallas import tpu as pltpu

_ROWS = 4096
_CHUNK = 8


def _body(mask_ref, o_fn_ref, g_fn_ref, m_fn_ref, o_fp_ref, g_fp_ref,
          m_fp_ref, res_ref, acc_ref):
    step = pl.program_id(0)

    @pl.when(step == 0)
    def _init():
        for i in range(4):
            acc_ref[i] = 0.0

    def chunk(j, carry):
        tot_fn, neg_fn, tot_fp, neg_fp = carry
        r = j * _CHUNK

        def branch(o_ref, g_ref, m_ref):
            o = o_ref[pl.ds(r, _CHUNK), :]
            g = g_ref[pl.ds(r, _CHUNK), :]
            m = m_ref[pl.ds(r, _CHUNK), :]
            om = 1.0 - o
            g1 = 1.0 - g
            g2 = g1 * g1
            g4 = g2 * g2
            isn = m == 0
            x = jnp.where(isn, om, o)
            xm = jnp.where(isn, o, om)
            w = xm * xm * jnp.where(isn, g4, 1.0)
            t = jnp.log(x) * w
            tn = jnp.where(isn, t, 0.0)
            return t, tn

        t1, tn1 = branch(o_fn_ref, g_fn_ref, m_fn_ref)
        t2, tn2 = branch(o_fp_ref, g_fp_ref, m_fp_ref)
        return (tot_fn + t1, neg_fn + tn1, tot_fp + t2, neg_fp + tn2)

    z = jnp.zeros((_CHUNK, 128), jnp.float32)
    tot_fn, neg_fn, tot_fp, neg_fp = jax.lax.fori_loop(
        0, _ROWS // _CHUNK, chunk, (z, z, z, z), unroll=8)

    acc_ref[0] = acc_ref[0] + jnp.sum(neg_fn)
    acc_ref[1] = acc_ref[1] + jnp.sum(tot_fn)
    acc_ref[2] = acc_ref[2] + jnp.sum(neg_fp)
    acc_ref[3] = acc_ref[3] + jnp.sum(tot_fp)

    @pl.when(step == pl.num_programs(0) - 1)
    def _fini():
        num_pos = jnp.sum(mask_ref[...])
        res_ref[0] = jnp.where(num_pos == 0.0, -acc_ref[0], -acc_ref[1])
        res_ref[1] = jnp.where(num_pos == 0.0, -acc_ref[2], -acc_ref[3])


def kernel(out, out_resi, target_resi, mask, negloss_fn_gt, fn_mask,
           negloss_fp_gt, fp_mask, wh_):
    B, C, H, W = out.shape
    R = B * C * H
    view = lambda a: a.reshape(R, W)
    grid = (R // _ROWS,)
    big = pl.BlockSpec((_ROWS, W), lambda i: (i, 0))
    res = pl.pallas_call(
        _body,
        grid=grid,
        in_specs=[
            pl.BlockSpec(mask.shape, lambda i: (0, 0)),
            big, big, big, big, big, big,
        ],
        out_specs=pl.BlockSpec(memory_space=pltpu.SMEM),
        out_shape=jax.ShapeDtypeStruct((2,), jnp.float32),
        scratch_shapes=[pltpu.SMEM((4,), jnp.float32)],
    )(mask, view(out), view(negloss_fn_gt), view(fn_mask),
      view(out_resi), view(negloss_fp_gt), view(fp_mask))
    return res[0], res[1]
